# Initial kernel scaffold; baseline (speedup 1.0000x reference)
import jax
import jax.numpy as jnp
from jax.experimental import pallas as pl


def kernel(x_nchw, weight, bias):
    raise NotImplementedError("write your pallas kernel here")


# speedup vs baseline: 1.0916x; 1.0916x over previous
import jax
import jax.numpy as jnp
from jax.experimental import pallas as pl
from jax.experimental.pallas import tpu as pltpu


def _round_up(x: int, m: int) -> int:
    return ((x + m - 1) // m) * m


def _fused_fc_kernel(x_ref, w_ref, b_ref, out1_ref, out3_ref):
    x = x_ref[...]
    out1_ref[...] = x
    acc = jnp.dot(x.astype(jnp.bfloat16), w_ref[...],
                  preferred_element_type=jnp.float32)
    out3_ref[...] = (acc + b_ref[...]).astype(out3_ref.dtype)


@jax.jit
def kernel(x_nchw, weight, bias):
    n = x_nchw.shape[0]
    x_flat = jnp.reshape(x_nchw, (n, -1))
    num_ftrs = x_flat.shape[1]
    num_classes = weight.shape[0]
    out_dtype = x_flat.dtype

    w_t = jnp.transpose(weight).astype(jnp.bfloat16)
    b2d = bias.astype(jnp.float32).reshape(1, num_classes)

    tm = 256
    n_pad = _round_up(n, tm)
    x_p = x_flat if n_pad == n else jnp.pad(x_flat, ((0, n_pad - n), (0, 0)))

    out1_p, out3_p = pl.pallas_call(
        _fused_fc_kernel,
        out_shape=(
            jax.ShapeDtypeStruct((n_pad, num_ftrs), out_dtype),
            jax.ShapeDtypeStruct((n_pad, num_classes), out_dtype),
        ),
        grid=(n_pad // tm,),
        in_specs=[
            pl.BlockSpec((tm, num_ftrs), lambda i: (i, 0)),
            pl.BlockSpec((num_ftrs, num_classes), lambda i: (0, 0)),
            pl.BlockSpec((1, num_classes), lambda i: (0, 0)),
        ],
        out_specs=(
            pl.BlockSpec((tm, num_ftrs), lambda i: (i, 0)),
            pl.BlockSpec((tm, num_classes), lambda i: (i, 0)),
        ),
        compiler_params=pltpu.CompilerParams(
            dimension_semantics=("parallel",),
            vmem_limit_bytes=48 * 1024 * 1024,
        ),
    )(x_p, w_t, b2d)

    if n_pad == n:
        return out1_p, out3_p
    return out1_p[:n], out3_p[:n]


# speedup vs baseline: 1.1481x; 1.0518x over previous
import jax
import jax.numpy as jnp
from jax.experimental import pallas as pl
from jax.experimental.pallas import tpu as pltpu


def _round_up(x: int, m: int) -> int:
    return ((x + m - 1) // m) * m


def _fused_fc_kernel(x_ref, w_ref, b_ref, out1_ref, out3_ref):
    x = x_ref[...]
    out1_ref[...] = x
    acc = jnp.dot(x.astype(jnp.bfloat16), w_ref[...],
                  preferred_element_type=jnp.float32)
    out3_ref[...] = (acc + b_ref[...]).astype(out3_ref.dtype)


@jax.jit
def kernel(x_nchw, weight, bias):
    n = x_nchw.shape[0]
    x_flat = jnp.reshape(x_nchw, (n, -1))
    num_ftrs = x_flat.shape[1]
    num_classes = weight.shape[0]
    out_dtype = x_flat.dtype

    w_t = jnp.transpose(weight).astype(jnp.bfloat16)
    b2d = bias.astype(jnp.float32).reshape(1, num_classes)

    tm = 512
    n_pad = _round_up(n, tm)
    x_p = x_flat if n_pad == n else jnp.pad(x_flat, ((0, n_pad - n), (0, 0)))

    out1_p, out3_p = pl.pallas_call(
        _fused_fc_kernel,
        out_shape=(
            jax.ShapeDtypeStruct((n_pad, num_ftrs), out_dtype),
            jax.ShapeDtypeStruct((n_pad, num_classes), out_dtype),
        ),
        grid=(n_pad // tm,),
        in_specs=[
            pl.BlockSpec((tm, num_ftrs), lambda i: (i, 0)),
            pl.BlockSpec((num_ftrs, num_classes), lambda i: (0, 0)),
            pl.BlockSpec((1, num_classes), lambda i: (0, 0)),
        ],
        out_specs=(
            pl.BlockSpec((tm, num_ftrs), lambda i: (i, 0)),
            pl.BlockSpec((tm, num_classes), lambda i: (i, 0)),
        ),
        compiler_params=pltpu.CompilerParams(
            dimension_semantics=("parallel",),
            vmem_limit_bytes=48 * 1024 * 1024,
        ),
    )(x_p, w_t, b2d)

    if n_pad == n:
        return out1_p, out3_p
    return out1_p[:n], out3_p[:n]


# speedup vs baseline: 1.1933x; 1.0394x over previous
import jax
import jax.numpy as jnp
from jax.experimental import pallas as pl
from jax.experimental.pallas import tpu as pltpu


def _round_up(x: int, m: int) -> int:
    return ((x + m - 1) // m) * m


def _fused_fc_kernel(x_ref, w_ref, b_ref, out3_ref):
    x = x_ref[...]
    acc = jnp.dot(x.astype(jnp.bfloat16), w_ref[...],
                  preferred_element_type=jnp.float32)
    out3_ref[...] = (acc + b_ref[...]).astype(out3_ref.dtype)


@jax.jit
def kernel(x_nchw, weight, bias):
    n = x_nchw.shape[0]
    x_flat = jnp.reshape(x_nchw, (n, -1))
    num_ftrs = x_flat.shape[1]
    num_classes = weight.shape[0]
    out_dtype = x_flat.dtype

    w_t = jnp.transpose(weight).astype(jnp.bfloat16)
    b2d = bias.astype(jnp.float32).reshape(1, num_classes)

    tm = 512
    n_pad = _round_up(n, tm)
    x_p = x_flat if n_pad == n else jnp.pad(x_flat, ((0, n_pad - n), (0, 0)))

    out3_p = pl.pallas_call(
        _fused_fc_kernel,
        out_shape=jax.ShapeDtypeStruct((n_pad, num_classes), out_dtype),
        grid=(n_pad // tm,),
        in_specs=[
            pl.BlockSpec((tm, num_ftrs), lambda i: (i, 0)),
            pl.BlockSpec((num_ftrs, num_classes), lambda i: (0, 0)),
            pl.BlockSpec((1, num_classes), lambda i: (0, 0)),
        ],
        out_specs=pl.BlockSpec((tm, num_classes), lambda i: (i, 0)),
        compiler_params=pltpu.CompilerParams(
            dimension_semantics=("parallel",),
            vmem_limit_bytes=48 * 1024 * 1024,
        ),
    )(x_p, w_t, b2d)

    out1 = jnp.copy(x_flat)
    if n_pad == n:
        return out1, out3_p
    return out1, out3_p[:n]
